# baseline (speedup 1.0000x reference)
import functools

import jax
import jax.numpy as jnp
from jax import lax
from jax.experimental import pallas as pl
from jax.experimental.pallas import tpu as pltpu
from jax.experimental.pallas import tpu_sc as plsc

N_NODE = 100000
EMBED_DIM = 128
CHUNK = 128
NUM_WORKERS = 32
FULL_CHUNKS = N_NODE // CHUNK
TAIL = N_NODE - FULL_CHUNKS * CHUNK
TAIL_BASE = FULL_CHUNKS * CHUNK
MAX_SLOTS = 25
MAX_Z_ROWS = 37
NBUF = 2
CB = CHUNK * EMBED_DIM

_mesh = plsc.VectorSubcoreMesh(core_axis_name="c", subcore_axis_name="s")


@functools.partial(
    pl.kernel,
    mesh=_mesh,
    out_type=jax.ShapeDtypeStruct((N_NODE * EMBED_DIM,), jnp.float32),
    scratch_types=[
        pltpu.VMEM((MAX_SLOTS * CHUNK,), jnp.int32),
        pltpu.VMEM((NBUF * CB,), jnp.float32),
        pltpu.VMEM((MAX_Z_ROWS * EMBED_DIM,), jnp.float32),
        pltpu.SemaphoreType.DMA,
        pltpu.SemaphoreType.DMA,
        pltpu.SemaphoreType.DMA,
        pltpu.SemaphoreType.DMA,
    ],
)
def _embed_sc(z_hbm, w_hbm, out_hbm, idx_all, rbuf, w_t, isem, tsem, w0, w1):
    wsem = (w0, w1)
    wid = lax.axis_index("s") * 2 + lax.axis_index("c")

    preload = [
        pltpu.async_copy(
            z_hbm.at[pl.ds((wid + s * NUM_WORKERS) * CHUNK, CHUNK)],
            idx_all.at[pl.ds(s * CHUNK, CHUNK)], isem)
        for s in range(MAX_SLOTS - 1)
    ]

    pltpu.sync_copy(w_hbm, w_t)

    @pl.when(wid < FULL_CHUNKS - (MAX_SLOTS - 1) * NUM_WORKERS)
    def _():
        s = MAX_SLOTS - 1
        pltpu.sync_copy(
            z_hbm.at[pl.ds((wid + s * NUM_WORKERS) * CHUNK, CHUNK)],
            idx_all.at[pl.ds(s * CHUNK, CHUNK)])

    for cp in preload:
        cp.wait()

    def expand(s, b, n_rows):
        def group_body(g, carry):
            zv = idx_all[pl.ds(s * CHUNK + g * 16, 16)]
            for r in range(16):
                off = zv[r] * EMBED_DIM
                dst = b * CB + (g * 16 + r) * EMBED_DIM
                for j in range(EMBED_DIM // 16):
                    rbuf[pl.ds(dst + j * 16, 16)] = w_t[pl.ds(off + j * 16, 16)]
            return carry

        lax.fori_loop(0, n_rows // 16, group_body, 0)

    def w_start(s, b):
        return pltpu.async_copy(
            rbuf.at[pl.ds(b * CB, CB)],
            out_hbm.at[pl.ds((wid + s * NUM_WORKERS) * CB, CB)], wsem[b])

    def w_wait(b):
        pltpu.make_async_copy(
            rbuf.at[pl.ds(b * CB, CB)], out_hbm.at[pl.ds(0, CB)],
            wsem[b]).wait()

    for s in range(NBUF):
        expand(s, s, CHUNK)
        w_start(s, s)

    def loop_body(it, carry):
        for j in range(NBUF):
            b = j
            s = NBUF + it * NBUF + j
            w_wait(b)
            expand(s, b, CHUNK)
            w_start(s, b)
        return carry

    lax.fori_loop(0, (MAX_SLOTS - 1 - NBUF) // NBUF, loop_body, 0)

    @pl.when(wid < FULL_CHUNKS - (MAX_SLOTS - 1) * NUM_WORKERS)
    def _():
        w_wait(0)
        expand(MAX_SLOTS - 1, 0, CHUNK)
        w_start(MAX_SLOTS - 1, 0)

    w_wait(0)
    w_wait(1)

    @pl.when(wid == NUM_WORKERS - 1)
    def _():
        pltpu.sync_copy(z_hbm.at[pl.ds(TAIL_BASE, TAIL)],
                        idx_all.at[pl.ds(0, TAIL)])
        expand(0, 0, TAIL)
        pltpu.async_copy(
            rbuf.at[pl.ds(0, TAIL * EMBED_DIM)],
            out_hbm.at[pl.ds(TAIL_BASE * EMBED_DIM, TAIL * EMBED_DIM)],
            tsem).wait()


def kernel(z, z_embed_weight):
    out = _embed_sc(z.astype(jnp.int32), z_embed_weight.reshape(-1))
    return out.reshape(N_NODE, EMBED_DIM)


# speedup vs baseline: 2.9394x; 2.9394x over previous
import functools

import jax
import jax.numpy as jnp
from jax import lax
from jax.experimental import pallas as pl
from jax.experimental.pallas import tpu as pltpu
from jax.experimental.pallas import tpu_sc as plsc

N_NODE = 100000
EMBED_DIM = 128
CHUNK = 128
NUM_WORKERS = 32
FULL_CHUNKS = N_NODE // CHUNK
TAIL = N_NODE - FULL_CHUNKS * CHUNK
TAIL_BASE = FULL_CHUNKS * CHUNK
MAX_SLOTS = 25
MAX_Z_ROWS = 37
NBUF = 6

_mesh = plsc.VectorSubcoreMesh(core_axis_name="c", subcore_axis_name="s")


@functools.partial(
    pl.kernel,
    mesh=_mesh,
    out_type=jax.ShapeDtypeStruct((N_NODE, EMBED_DIM), jnp.float32),
    scratch_types=[
        pltpu.VMEM((MAX_SLOTS * CHUNK,), jnp.int32),
        pltpu.VMEM((NBUF * CHUNK, EMBED_DIM), jnp.float32),
        pltpu.VMEM((TAIL,), jnp.int32),
        pltpu.VMEM((TAIL, EMBED_DIM), jnp.float32),
        pltpu.VMEM_SHARED((MAX_Z_ROWS, EMBED_DIM), jnp.float32),
        pltpu.VMEM_SHARED((MAX_Z_ROWS, EMBED_DIM), jnp.float32),
        pltpu.SemaphoreType.DMA,
        pltpu.SemaphoreType.DMA,
        pltpu.SemaphoreType.DMA,
        pltpu.SemaphoreType.DMA,
        pltpu.SemaphoreType.DMA,
        pltpu.SemaphoreType.DMA,
        pltpu.SemaphoreType.DMA,
        pltpu.SemaphoreType.DMA,
        pltpu.SemaphoreType.DMA,
        pltpu.SemaphoreType.DMA,
        pltpu.SemaphoreType.DMA,
        pltpu.SemaphoreType.DMA,
        pltpu.SemaphoreType.DMA,
        pltpu.SemaphoreType.DMA,
    ],
)
def _embed_sc(z_hbm, w_hbm, out_hbm, idx_all, rbuf, idx_t, rows_t, w_sh,
              w_sh2, isem, tsem, g0, g1, g2, g3, g4, g5, w0, w1, w2, w3, w4,
              w5):
    gsem = (g0, g1, g2, g3, g4, g5)
    wsem = (w0, w1, w2, w3, w4, w5)
    wid = lax.axis_index("s") * 2 + lax.axis_index("c")

    preload = [
        pltpu.async_copy(
            z_hbm.at[pl.ds((wid + s * NUM_WORKERS) * CHUNK, CHUNK)],
            idx_all.at[pl.ds(s * CHUNK, CHUNK)], isem)
        for s in range(MAX_SLOTS - 1)
    ]

    @pl.when(lax.axis_index("s") == 0)
    def _():
        pltpu.sync_copy(w_hbm, w_sh)

    @pl.when(lax.axis_index("s") == 1)
    def _():
        pltpu.sync_copy(w_hbm, w_sh2)

    @pl.when(wid < FULL_CHUNKS - (MAX_SLOTS - 1) * NUM_WORKERS)
    def _():
        s = MAX_SLOTS - 1
        pltpu.sync_copy(
            z_hbm.at[pl.ds((wid + s * NUM_WORKERS) * CHUNK, CHUNK)],
            idx_all.at[pl.ds(s * CHUNK, CHUNK)])

    for cp in preload:
        cp.wait()

    plsc.subcore_barrier()

    def buf(b):
        return rbuf.at[pl.ds(b * CHUNK, CHUNK)]

    def g_start(s, b, par):
        tab = w_sh if par == 0 else w_sh2
        return pltpu.async_copy(
            tab.at[idx_all.at[pl.ds(s * CHUNK, CHUNK)]], buf(b), gsem[b])

    def g_wait(b):
        pltpu.make_async_copy(
            out_hbm.at[pl.ds(0, CHUNK)], buf(b), gsem[b]).wait()

    def w_start(s, b):
        return pltpu.async_copy(
            buf(b), out_hbm.at[pl.ds((wid + s * NUM_WORKERS) * CHUNK, CHUNK)],
            wsem[b])

    def w_wait(b):
        pltpu.make_async_copy(
            buf(b), out_hbm.at[pl.ds(0, CHUNK)], wsem[b]).wait()

    for s in range(NBUF - 1):
        g_start(s, s, s % 2)

    for s in range(NBUF - 1):
        g_wait(s)
        w_start(s, s)
        if s >= 1:
            w_wait(s - 1)
        g_start(s + NBUF - 1, (s + NBUF - 1) % NBUF, (s + NBUF - 1) % 2)

    def loop_body(it, carry):
        for j in range(NBUF):
            b = (NBUF - 1 + j) % NBUF
            nb = (NBUF - 2 + j) % NBUF
            s = (NBUF - 1) + it * NBUF + j
            g_wait(b)
            w_start(s, b)

            @pl.when(wid + (s + NBUF - 1) * NUM_WORKERS < FULL_CHUNKS)
            def _():
                w_wait(nb)
                g_start(s + NBUF - 1, nb, (NBUF - 1 + j + 1) % 2)

        return carry

    lax.fori_loop(0, (23 - (NBUF - 1)) // NBUF, loop_body, 0)

    g_wait(23 % NBUF)
    w_start(23, 23 % NBUF)

    @pl.when(wid < FULL_CHUNKS - (MAX_SLOTS - 1) * NUM_WORKERS)
    def _():
        g_wait(24 % NBUF)
        w_start(24, 24 % NBUF)

    @pl.when(wid == NUM_WORKERS - 1)
    def _():
        pltpu.sync_copy(z_hbm.at[pl.ds(TAIL_BASE, TAIL)], idx_t)
        pltpu.async_copy(w_sh.at[idx_t], rows_t, tsem).wait()
        pltpu.sync_copy(rows_t, out_hbm.at[pl.ds(TAIL_BASE, TAIL)])

    for b in range(NBUF):
        w_wait(b)


def kernel(z, z_embed_weight):
    return _embed_sc(z.astype(jnp.int32), z_embed_weight)
